# Initial kernel scaffold; baseline (speedup 1.0000x reference)
import jax
import jax.numpy as jnp
from jax.experimental import pallas as pl


def kernel(h, edge_index, W1, b1, W2, b2):
    raise NotImplementedError("write your pallas kernel here")


# speedup vs baseline: 5.8309x; 5.8309x over previous
import jax
import jax.numpy as jnp
from jax import lax
from jax.experimental import pallas as pl
from jax.experimental.pallas import tpu as pltpu
from jax.experimental.pallas import tpu_sc as plsc

N = 10000
D = 128
DH = D // 2
E = 320000
NC = 2
NS = 16
K = 128
CHUNKS = -(-E // (NS * K))
EPW = CHUNKS * K
EPAD = EPW * NS
NPAD = 10240
RPT = NPAD // NS
BN = 256
f32 = jnp.float32


def _make_sc_agg(with_deg):
    mesh = plsc.VectorSubcoreMesh(
        core_axis_name="c", subcore_axis_name="s",
        num_cores=NC, num_subcores=NS)
    if with_deg:
        out_type = (jax.ShapeDtypeStruct((NC, NPAD, DH), f32),
                    jax.ShapeDtypeStruct((NPAD, 16), f32))
    else:
        out_type = jax.ShapeDtypeStruct((NC, NPAD, DH), f32)
    scratch = [
        pltpu.VMEM((CHUNKS, K), jnp.int32),
        pltpu.VMEM((CHUNKS, K), jnp.int32),
        pltpu.VMEM((K, DH), f32),
        pltpu.VMEM((K, 16), f32),
        pltpu.VMEM_SHARED((NPAD, DH), f32),
        pltpu.VMEM_SHARED((NPAD, 16), f32),
        pltpu.SemaphoreType.DMA,
    ]

    def body(xa_hbm, xb_hbm, src_hbm, dst_hbm, zrow_hbm, z16_hbm, ones_hbm,
             *rest):
        if with_deg:
            acc_out, deg_out = rest[0], rest[1]
            rest = rest[2:]
        else:
            acc_out = rest[0]
            rest = rest[1:]
        src_v, dst_v, rows_v, ones_v, acc_sh, deg_sh, sem = rest
        c = lax.axis_index("c")
        s = lax.axis_index("s")
        r0 = s * RPT
        pltpu.sync_copy(zrow_hbm, acc_sh.at[pl.ds(r0, RPT)])
        if with_deg:
            @pl.when(c == 0)
            def _():
                pltpu.sync_copy(z16_hbm, deg_sh.at[pl.ds(r0, RPT)])
                pltpu.sync_copy(ones_hbm, ones_v)
        pltpu.sync_copy(src_hbm.at[s], src_v)
        pltpu.sync_copy(dst_hbm.at[s], dst_v)
        plsc.subcore_barrier()

        def step(j, carry):
            @pl.when(c == 0)
            def _():
                pltpu.async_copy(xa_hbm.at[src_v.at[j]], rows_v, sem).wait()

            @pl.when(c == 1)
            def _():
                pltpu.async_copy(xb_hbm.at[src_v.at[j]], rows_v, sem).wait()

            pltpu.sync_copy(rows_v, acc_sh.at[dst_v.at[j]], add=True)
            if with_deg:
                @pl.when(c == 0)
                def _():
                    pltpu.sync_copy(ones_v, deg_sh.at[dst_v.at[j]], add=True)
            return carry

        lax.fori_loop(0, CHUNKS, step, 0)
        plsc.subcore_barrier()
        pltpu.sync_copy(acc_sh.at[pl.ds(r0, RPT)],
                        acc_out.at[c, pl.ds(r0, RPT)])
        if with_deg:
            @pl.when(c == 0)
            def _():
                pltpu.sync_copy(deg_sh.at[pl.ds(r0, RPT)],
                                deg_out.at[pl.ds(r0, RPT)])

    return pl.kernel(body, out_type=out_type, mesh=mesh,
                     scratch_types=scratch,
                     compiler_params=pltpu.CompilerParams(
                         use_tc_tiling_on_sc=False))


_sc_agg_deg = _make_sc_agg(True)
_sc_agg = _make_sc_agg(False)


def _make_tc_layer(split_out):
    def body(acc_ref, deg_ref, w_ref, b_ref, *out_refs):
        a = jnp.concatenate([acc_ref[0], acc_ref[1]], axis=1)
        dg = jnp.maximum(deg_ref[:, 0:1], 1.0)
        agg = a / dg
        y = lax.dot_general(agg, w_ref[...], (((1,), (1,)), ((), ())),
                            preferred_element_type=f32) + b_ref[...]
        y = jnp.where(y > 0.0, y, jnp.exp(y) - 1.0)
        if split_out:
            out_refs[0][...] = y[:, :DH]
            out_refs[1][...] = y[:, DH:]
        else:
            out_refs[0][...] = y

    if split_out:
        out_shape = (jax.ShapeDtypeStruct((NPAD, DH), f32),) * 2
        out_specs = (pl.BlockSpec((BN, DH), lambda i: (i, 0)),) * 2
    else:
        out_shape = jax.ShapeDtypeStruct((NPAD, D), f32)
        out_specs = pl.BlockSpec((BN, D), lambda i: (i, 0))
    return pl.pallas_call(
        body,
        grid=(NPAD // BN,),
        in_specs=[
            pl.BlockSpec((NC, BN, DH), lambda i: (0, i, 0)),
            pl.BlockSpec((BN, 16), lambda i: (i, 0)),
            pl.BlockSpec((D, D), lambda i: (0, 0)),
            pl.BlockSpec((1, D), lambda i: (0, 0)),
        ],
        out_specs=out_specs,
        out_shape=out_shape,
    )


_tc_mid = _make_tc_layer(True)
_tc_last = _make_tc_layer(False)


def kernel(h, edge_index, W1, b1, W2, b2):
    ei = edge_index.astype(jnp.int32)
    pad = EPAD - E
    src_p = jnp.concatenate(
        [ei[0], jnp.zeros((pad,), jnp.int32)]).reshape(NS, CHUNKS, K)
    dst_p = jnp.concatenate(
        [ei[1], jnp.full((pad,), NPAD - 1, jnp.int32)]).reshape(NS, CHUNKS, K)
    h_p = jnp.zeros((NPAD, D), f32).at[:N].set(h)
    ha, hb = h_p[:, :DH], h_p[:, DH:]
    zrow = jnp.zeros((RPT, DH), f32)
    z16 = jnp.zeros((RPT, 16), f32)
    ones = jnp.ones((K, 16), f32)

    acc1, deg = _sc_agg_deg(ha, hb, src_p, dst_p, zrow, z16, ones)
    x1a, x1b = _tc_mid(acc1, deg, W1, b1.reshape(1, D))
    acc2 = _sc_agg(x1a, x1b, src_p, dst_p, zrow, z16, ones)
    x2 = _tc_last(acc2, deg, W2, b2.reshape(1, D))
    return x2[:N]


# speedup vs baseline: 6.6982x; 1.1487x over previous
import jax
import jax.numpy as jnp
from jax import lax
from jax.experimental import pallas as pl
from jax.experimental.pallas import tpu as pltpu
from jax.experimental.pallas import tpu_sc as plsc

N = 10000
D = 128
DH = D // 2
E = 320000
NC = 2
NS = 16
K = 128
CHUNKS = 158
PAIRS = CHUNKS // 2
EPW = CHUNKS * K
EPAD = EPW * NS
NPAD = 10240
RPT = NPAD // NS
BN = 256
f32 = jnp.float32


def _make_sc_agg(with_deg):
    mesh = plsc.VectorSubcoreMesh(
        core_axis_name="c", subcore_axis_name="s",
        num_cores=NC, num_subcores=NS)
    if with_deg:
        out_type = (jax.ShapeDtypeStruct((NC, NPAD, DH), f32),
                    jax.ShapeDtypeStruct((NPAD, 16), f32))
    else:
        out_type = jax.ShapeDtypeStruct((NC, NPAD, DH), f32)
    scratch = [
        pltpu.VMEM((CHUNKS, K), jnp.int32),
        pltpu.VMEM((CHUNKS, K), jnp.int32),
        pltpu.VMEM((K, DH), f32),
        pltpu.VMEM((K, DH), f32),
        pltpu.VMEM((K, 16), f32),
        pltpu.VMEM_SHARED((NPAD, DH), f32),
        pltpu.VMEM_SHARED((NPAD, 16), f32),
        pltpu.SemaphoreType.DMA,
        pltpu.SemaphoreType.DMA,
    ]

    def body(xa_hbm, xb_hbm, src_hbm, dst_hbm, zrow_hbm, z16_hbm, ones_hbm,
             *rest):
        if with_deg:
            acc_out, deg_out = rest[0], rest[1]
            rest = rest[2:]
        else:
            acc_out = rest[0]
            rest = rest[1:]
        src_v, dst_v, rows_a, rows_b, ones_v, acc_sh, deg_sh, sem_a, sem_b = rest
        c = lax.axis_index("c")
        s = lax.axis_index("s")
        r0 = s * RPT
        pltpu.sync_copy(zrow_hbm, acc_sh.at[pl.ds(r0, RPT)])
        if with_deg:
            @pl.when(c == 0)
            def _():
                pltpu.sync_copy(z16_hbm, deg_sh.at[pl.ds(r0, RPT)])
                pltpu.sync_copy(ones_hbm, ones_v)
        pltpu.sync_copy(src_hbm.at[s], src_v)
        pltpu.sync_copy(dst_hbm.at[s], dst_v)
        plsc.subcore_barrier()

        def gather(j, buf, sm):
            @pl.when(c == 0)
            def _():
                pltpu.async_copy(xa_hbm.at[src_v.at[j]], buf, sm)

            @pl.when(c == 1)
            def _():
                pltpu.async_copy(xb_hbm.at[src_v.at[j]], buf, sm)

        def gwait(buf, sm):
            pltpu.make_async_copy(xa_hbm.at[src_v.at[0]], buf, sm).wait()

        def scat(j, buf):
            pltpu.sync_copy(buf, acc_sh.at[dst_v.at[j]], add=True)
            if with_deg:
                @pl.when(c == 0)
                def _():
                    pltpu.sync_copy(ones_v, deg_sh.at[dst_v.at[j]], add=True)

        gather(0, rows_a, sem_a)

        def pair(i, carry):
            j0 = 2 * i
            gather(j0 + 1, rows_b, sem_b)
            gwait(rows_a, sem_a)
            scat(j0, rows_a)

            @pl.when(i < PAIRS - 1)
            def _():
                gather(j0 + 2, rows_a, sem_a)

            gwait(rows_b, sem_b)
            scat(j0 + 1, rows_b)
            return carry

        lax.fori_loop(0, PAIRS, pair, 0)
        plsc.subcore_barrier()
        pltpu.sync_copy(acc_sh.at[pl.ds(r0, RPT)],
                        acc_out.at[c, pl.ds(r0, RPT)])
        if with_deg:
            @pl.when(c == 0)
            def _():
                pltpu.sync_copy(deg_sh.at[pl.ds(r0, RPT)],
                                deg_out.at[pl.ds(r0, RPT)])

    return pl.kernel(body, out_type=out_type, mesh=mesh,
                     scratch_types=scratch,
                     compiler_params=pltpu.CompilerParams(
                         use_tc_tiling_on_sc=False))


_sc_agg_deg = _make_sc_agg(True)
_sc_agg = _make_sc_agg(False)


def _make_tc_layer(split_out):
    def body(acc_ref, deg_ref, w_ref, b_ref, *out_refs):
        a = jnp.concatenate([acc_ref[0], acc_ref[1]], axis=1)
        dg = jnp.maximum(deg_ref[:, 0:1], 1.0)
        agg = a / dg
        y = lax.dot_general(agg, w_ref[...], (((1,), (1,)), ((), ())),
                            preferred_element_type=f32) + b_ref[...]
        y = jnp.where(y > 0.0, y, jnp.exp(y) - 1.0)
        if split_out:
            out_refs[0][...] = y[:, :DH]
            out_refs[1][...] = y[:, DH:]
        else:
            out_refs[0][...] = y

    if split_out:
        out_shape = (jax.ShapeDtypeStruct((NPAD, DH), f32),) * 2
        out_specs = (pl.BlockSpec((BN, DH), lambda i: (i, 0)),) * 2
    else:
        out_shape = jax.ShapeDtypeStruct((NPAD, D), f32)
        out_specs = pl.BlockSpec((BN, D), lambda i: (i, 0))
    return pl.pallas_call(
        body,
        grid=(NPAD // BN,),
        in_specs=[
            pl.BlockSpec((NC, BN, DH), lambda i: (0, i, 0)),
            pl.BlockSpec((BN, 16), lambda i: (i, 0)),
            pl.BlockSpec((D, D), lambda i: (0, 0)),
            pl.BlockSpec((1, D), lambda i: (0, 0)),
        ],
        out_specs=out_specs,
        out_shape=out_shape,
    )


_tc_mid = _make_tc_layer(True)
_tc_last = _make_tc_layer(False)


def kernel(h, edge_index, W1, b1, W2, b2):
    ei = edge_index.astype(jnp.int32)
    pad = EPAD - E
    src_p = jnp.concatenate(
        [ei[0], jnp.zeros((pad,), jnp.int32)]).reshape(NS, CHUNKS, K)
    dst_p = jnp.concatenate(
        [ei[1], jnp.full((pad,), NPAD - 1, jnp.int32)]).reshape(NS, CHUNKS, K)
    h_p = jnp.zeros((NPAD, D), f32).at[:N].set(h)
    ha, hb = h_p[:, :DH], h_p[:, DH:]
    zrow = jnp.zeros((RPT, DH), f32)
    z16 = jnp.zeros((RPT, 16), f32)
    ones = jnp.ones((K, 16), f32)

    acc1, deg = _sc_agg_deg(ha, hb, src_p, dst_p, zrow, z16, ones)
    x1a, x1b = _tc_mid(acc1, deg, W1, b1.reshape(1, D))
    acc2 = _sc_agg(x1a, x1b, src_p, dst_p, zrow, z16, ones)
    x2 = _tc_last(acc2, deg, W2, b2.reshape(1, D))
    return x2[:N]
